# baseline (speedup 1.0000x reference)
import jax
import jax.numpy as jnp
from jax import lax
from jax.experimental import pallas as pl
from jax.experimental.pallas import tpu as pltpu
from jax.experimental.pallas import tpu_sc as plsc

N_NODES = 10000
IN_DIM = 128
HID_DIM = 256
OUT_DIM = 128

NPAD = 10240
N_EDGES2 = 640000
N_TILES = 32
CHUNK = 32
BLK_CH = 40
E2_PAD = 655360
N_BLKS_A = 32
N_BLKS_D = 16
ROWS_PER_TILE = NPAD // 16

HALF = 5000
ACC_ROWS = 5120
ACC_RPT = ACC_ROWS // 16
TBL_RPT = 632

_MESH = dict(core_axis_name="c", subcore_axis_name="s", num_cores=2,
             num_subcores=16)


def _sc_agg_body(table, srcr, dstrc, zrows,
                 s_out,
                 sidx, didx, r0, tbl_sh, acc_sh, g0):
    c = lax.axis_index("c")
    s = lax.axis_index("s")

    @pl.when(s < 15)
    def _():
        pltpu.sync_copy(table.at[pl.ds(s * TBL_RPT, TBL_RPT)],
                        tbl_sh.at[pl.ds(s * TBL_RPT, TBL_RPT)])

    @pl.when(s == 15)
    def _():
        pltpu.sync_copy(table.at[pl.ds(15 * TBL_RPT, N_NODES - 15 * TBL_RPT)],
                        tbl_sh.at[pl.ds(15 * TBL_RPT, N_NODES - 15 * TBL_RPT)])

    pltpu.sync_copy(zrows.at[pl.ds(0, ACC_RPT)],
                    acc_sh.at[pl.ds(s * ACC_RPT, ACC_RPT)])
    plsc.subcore_barrier()

    def step(ch, inner):
        pltpu.sync_copy(srcr.at[s].at[ch], sidx)
        pltpu.sync_copy(dstrc.at[c].at[s].at[ch], didx)
        pltpu.async_copy(tbl_sh.at[sidx], r0, g0).wait()
        pltpu.sync_copy(r0, acc_sh.at[didx], add=True)
        return inner

    lax.fori_loop(0, N_BLKS_A * BLK_CH, step, 0)
    plsc.subcore_barrier()

    rs = pl.ds(s * ACC_RPT, ACC_RPT)
    pltpu.sync_copy(acc_sh.at[rs], s_out.at[c].at[rs])


def _sc_deg_body(dstr, zrows, ones128,
                 deg_out,
                 didx, ones_v, deg_sh, sem):
    c = lax.axis_index("c")
    s = lax.axis_index("s")
    wid = s * 2 + c
    k = 8

    pltpu.sync_copy(zrows, deg_sh.at[pl.ds(s * ROWS_PER_TILE, ROWS_PER_TILE)])
    pltpu.sync_copy(ones128, ones_v)
    plsc.subcore_barrier()

    def block(b, carry):
        pltpu.sync_copy(dstr.at[wid].at[b], didx)

        def grp(g, inner):
            for i in range(k):
                pltpu.async_copy(ones_v, deg_sh.at[didx.at[g * k + i]], sem,
                                 add=True)
            for i in range(k):
                pltpu.make_async_copy(ones_v, deg_sh.at[didx.at[g * k + i]],
                                      sem).wait()
            return inner

        lax.fori_loop(0, BLK_CH // k, grp, carry)
        return carry

    lax.fori_loop(0, N_BLKS_D, block, 0)
    plsc.subcore_barrier()

    rs = pl.ds(s * ROWS_PER_TILE, ROWS_PER_TILE)
    pltpu.sync_copy(deg_sh.at[rs], deg_out.at[c].at[rs])


_sc_agg = pl.kernel(
    _sc_agg_body,
    out_type=(jax.ShapeDtypeStruct((2, ACC_ROWS, 128), jnp.float32),),
    mesh=plsc.VectorSubcoreMesh(**_MESH),
    scratch_types=(
        pltpu.VMEM((CHUNK,), jnp.int32),
        pltpu.VMEM((CHUNK,), jnp.int32),
        pltpu.VMEM((CHUNK, 128), jnp.float32),
        pltpu.VMEM_SHARED((N_NODES, 128), jnp.float32),
        pltpu.VMEM_SHARED((ACC_ROWS, 128), jnp.float32),
        pltpu.SemaphoreType.DMA,
    ),
)

_sc_deg = pl.kernel(
    _sc_deg_body,
    out_type=(jax.ShapeDtypeStruct((2, NPAD, 128), jnp.float32),),
    mesh=plsc.VectorSubcoreMesh(**_MESH),
    scratch_types=(
        pltpu.VMEM((BLK_CH, CHUNK), jnp.int32),
        pltpu.VMEM((CHUNK, 128), jnp.float32),
        pltpu.VMEM_SHARED((NPAD, 128), jnp.float32),
        pltpu.SemaphoreType.DMA,
    ),
)


_R = 1000


def _clamp_body(dref, o):
    d = dref[...]
    for cc in range(2):
        base = cc * HALF
        inr = (d >= base) & (d < base + HALF)
        o[cc] = jnp.where(inr, d - base, HALF)


def _clamp(dflat):
    return pl.pallas_call(
        _clamp_body,
        out_shape=jax.ShapeDtypeStruct((2,) + dflat.shape, jnp.int32),
    )(dflat)


def _sget(s0, s1):
    side = pl.program_id(0) >= (HALF // _R)
    return jnp.where(side, s1[0], s0[0])


def _deg_factors(dref):
    d = dref[0][:, 0:1] + dref[1][:, 0:1]
    invd = 1.0 / jnp.maximum(d, 1.0)
    sself = 0.5 * d * invd
    hinv = 0.5 * invd
    hasdeg = (d > 0.0).astype(jnp.float32)
    return sself, hinv, hasdeg


def _t1_body(f, s0, s1, dref, w, b, o):
    sself, hinv, hasdeg = _deg_factors(dref)
    p = sself * f[...] + hinv * _sget(s0, s1)
    o[...] = jax.nn.relu(
        jnp.dot(p, w[...], preferred_element_type=jnp.float32) + hasdeg * b[...])


def _t2_body(h, w, b, o):
    o[...] = jnp.dot(h[...], w[...], preferred_element_type=jnp.float32) + b[...]


def _t3_body(z, s0, s1, dref, w, b, h2, z3):
    sself, hinv, _ = _deg_factors(dref)
    h = jax.nn.relu(sself * z[...] + hinv * _sget(s0, s1))
    h2[...] = h
    z3[...] = jnp.dot(h, w[...], preferred_element_type=jnp.float32) + b[...]


def _t4_body(z, s0, s1, dref, o):
    sself, hinv, _ = _deg_factors(dref)
    o[...] = jax.nn.relu(sself * z[...] + hinv * _sget(s0, s1))


def _row_spec(cols):
    return pl.BlockSpec((_R, cols), lambda i: (i, 0))


def _part_spec(cols):
    return pl.BlockSpec((2, _R, cols), lambda i: (0, i, 0))


def _full_spec(r, c):
    return pl.BlockSpec((r, c), lambda i: (0, 0))


_NB0 = HALF // _R


def _acc0_spec():
    return pl.BlockSpec((1, _R, 128), lambda i: (0, jnp.minimum(i, _NB0 - 1), 0))


def _acc1_spec():
    return pl.BlockSpec((1, _R, 128), lambda i: (1, jnp.maximum(i - _NB0, 0), 0))


def _t1(f, sa, degp, w1, b1):
    return pl.pallas_call(
        _t1_body,
        grid=(N_NODES // _R,),
        in_specs=[_row_spec(IN_DIM), _acc0_spec(), _acc1_spec(),
                  _part_spec(128),
                  _full_spec(IN_DIM, HID_DIM), _full_spec(1, HID_DIM)],
        out_specs=_row_spec(HID_DIM),
        out_shape=jax.ShapeDtypeStruct((N_NODES, HID_DIM), jnp.float32),
    )(f, sa, sa, degp, w1, b1)


def _t2(h, w2, b2):
    return pl.pallas_call(
        _t2_body,
        grid=(N_NODES // _R,),
        in_specs=[_row_spec(HID_DIM), _full_spec(HID_DIM, OUT_DIM),
                  _full_spec(1, OUT_DIM)],
        out_specs=_row_spec(OUT_DIM),
        out_shape=jax.ShapeDtypeStruct((N_NODES, OUT_DIM), jnp.float32),
    )(h, w2, b2)


def _t3(z2, sb, degp, w3, b3):
    return pl.pallas_call(
        _t3_body,
        grid=(N_NODES // _R,),
        in_specs=[_row_spec(OUT_DIM), _acc0_spec(), _acc1_spec(),
                  _part_spec(128),
                  _full_spec(OUT_DIM, OUT_DIM), _full_spec(1, OUT_DIM)],
        out_specs=[_row_spec(OUT_DIM), _row_spec(OUT_DIM)],
        out_shape=[jax.ShapeDtypeStruct((N_NODES, OUT_DIM), jnp.float32),
                   jax.ShapeDtypeStruct((N_NODES, OUT_DIM), jnp.float32)],
    )(z2, sb, sb, degp, w3, b3)


def _t4(z3, sc, degp):
    return pl.pallas_call(
        _t4_body,
        grid=(N_NODES // _R,),
        in_specs=[_row_spec(OUT_DIM), _acc0_spec(), _acc1_spec(),
                  _part_spec(128)],
        out_specs=_row_spec(OUT_DIM),
        out_shape=jax.ShapeDtypeStruct((N_NODES, OUT_DIM), jnp.float32),
    )(z3, sc, sc, degp)


def kernel(features, edge_index, W1, b1, W2, b2, W3, b3):
    src = edge_index[0].astype(jnp.int32)
    dst = edge_index[1].astype(jnp.int32)
    src2 = jnp.concatenate([src, dst])
    dst2 = jnp.concatenate([dst, src])
    pad = E2_PAD - N_EDGES2
    src2p = jnp.pad(src2, (0, pad))
    dst2p = jnp.pad(dst2, (0, pad), constant_values=NPAD - 1)
    srcr = src2p.reshape(16, N_BLKS_A * BLK_CH, CHUNK)
    dstrc = _clamp(dst2p.reshape(E2_PAD // 128, 128)) \
        .reshape(2, 16, N_BLKS_A * BLK_CH, CHUNK)
    dstr = dst2p.reshape(N_TILES, N_BLKS_D, BLK_CH, CHUNK)

    zrows = jnp.zeros((ROWS_PER_TILE, 128), jnp.float32)
    ones128 = jnp.ones((CHUNK, 128), jnp.float32)

    b1r = b1.reshape(1, HID_DIM)
    b2r = b2.reshape(1, OUT_DIM)
    b3r = b3.reshape(1, OUT_DIM)

    (degp,) = _sc_deg(dstr, zrows, ones128)
    (sa,) = _sc_agg(features, srcr, dstrc, zrows)
    h1 = _t1(features, sa, degp, W1, b1r)
    z2 = _t2(h1, W2, b2r)
    (sb,) = _sc_agg(z2, srcr, dstrc, zrows)
    h2, z3 = _t3(z2, sb, degp, W3, b3r)
    (sc,) = _sc_agg(z3, srcr, dstrc, zrows)
    logits = _t4(z3, sc, degp)
    return (h2, logits)
